# Initial kernel scaffold; baseline (speedup 1.0000x reference)
import jax
import jax.numpy as jnp
from jax.experimental import pallas as pl


def kernel(x, edge_index, W1, b1, ln_g, ln_b, W2, b2):
    raise NotImplementedError("write your pallas kernel here")


# speedup vs baseline: 24.8700x; 24.8700x over previous
import functools

import jax
import jax.numpy as jnp
from jax import lax
from jax.experimental import pallas as pl
from jax.experimental.pallas import tpu as pltpu
from jax.experimental.pallas import tpu_sc as plsc

N = 100000
E = 1600000
IN_CH = 128
HID = 32
EMB = 32
HALF = 16

NC = 2
NS = 16
NP = 100352
EP = 1605632
ER = EP // 128
ROWS_PER_TILE = ER // NS
GROUPS = ROWS_PER_TILE // 8
DEG_ROWS_PER_TILE = ER // (NC * NS)
DEG_GROUPS = DEG_ROWS_PER_TILE // 8
NODE_PER_TILE = NP // NS
NODE_CHUNK = 896
NODE_CHUNKS = NODE_PER_TILE // NODE_CHUNK

BLK = 3136
TC_GRID = NP // BLK

_mesh = plsc.VectorSubcoreMesh(core_axis_name="c", subcore_axis_name="s")


def _encoder_body(x_ref, w1_ref, b1_ref, lg_ref, lb_ref, w2_ref, b2_ref, o_ref):
    x = x_ref[...]
    h = lax.dot_general(x, w1_ref[...], (((1,), (1,)), ((), ())),
                        preferred_element_type=jnp.float32,
                        precision=lax.Precision.HIGHEST)
    h = h + b1_ref[...]
    mu = jnp.mean(h, axis=-1, keepdims=True)
    var = jnp.mean((h - mu) ** 2, axis=-1, keepdims=True)
    h = (h - mu) * lax.rsqrt(var + 1e-06) * lg_ref[...] + lb_ref[...]
    h = jnp.maximum(h, 0.0)
    h = lax.dot_general(h, w2_ref[...], (((1,), (1,)), ((), ())),
                        preferred_element_type=jnp.float32,
                        precision=lax.Precision.HIGHEST)
    h = h + b2_ref[...]
    nrm = jnp.sqrt(jnp.sum(h * h, axis=-1, keepdims=True))
    o_ref[...] = h / jnp.maximum(nrm, 1e-12)


def _encoder(xp, W1, b1, ln_g, ln_b, W2, b2):
    full = lambda i: (0, 0)
    return pl.pallas_call(
        _encoder_body,
        grid=(TC_GRID,),
        in_specs=[
            pl.BlockSpec((BLK, IN_CH), lambda i: (i, 0)),
            pl.BlockSpec((HID, IN_CH), full),
            pl.BlockSpec((1, HID), full),
            pl.BlockSpec((1, HID), full),
            pl.BlockSpec((1, HID), full),
            pl.BlockSpec((EMB, HID), full),
            pl.BlockSpec((1, EMB), full),
        ],
        out_specs=pl.BlockSpec((BLK, EMB), lambda i: (i, 0)),
        out_shape=jax.ShapeDtypeStruct((NP, EMB), jnp.float32),
    )(xp, W1, b1.reshape(1, -1), ln_g.reshape(1, -1), ln_b.reshape(1, -1),
      W2, b2.reshape(1, -1))


def _deg_body(dst_hbm, out_hbm, idx_d, ones_v, stage, deg_acc):
    c = lax.axis_index("c")
    s = lax.axis_index("s")
    one16 = jnp.ones((16,), jnp.float32)
    zero16 = jnp.zeros((16,), jnp.float32)
    for i in range(8):
        ones_v[pl.ds(i * 16, 16)] = one16

    def _z(i, _):
        stage[pl.ds(i * 16, 16)] = zero16
        return _
    lax.fori_loop(0, NODE_PER_TILE // 16, _z, None)
    pltpu.sync_copy(stage, deg_acc.at[pl.ds(s * NODE_PER_TILE, NODE_PER_TILE)])
    plsc.subcore_barrier()

    row0 = c * (ER // 2) + s * DEG_ROWS_PER_TILE

    def _grp(g, _):
        pltpu.sync_copy(dst_hbm.at[pl.ds(row0 + g * 8, 8)], idx_d)
        for j in range(8):
            pltpu.sync_copy(ones_v.at[pl.ds(0, 128)],
                            deg_acc.at[idx_d.at[j]], add=True)
        return _
    lax.fori_loop(0, DEG_GROUPS, _grp, None)
    plsc.subcore_barrier()

    off = s * NODE_PER_TILE
    pltpu.sync_copy(deg_acc.at[pl.ds(off, NODE_PER_TILE)], stage)
    pltpu.sync_copy(stage, out_hbm.at[pl.ds(c * NP + off, NODE_PER_TILE)])


_deg_call = pl.kernel(
    _deg_body,
    out_type=jax.ShapeDtypeStruct((NC * NP,), jnp.float32),
    mesh=_mesh,
    scratch_types=[
        pltpu.VMEM((8, 128), jnp.int32),
        pltpu.VMEM((128,), jnp.float32),
        pltpu.VMEM((NODE_PER_TILE,), jnp.float32),
        pltpu.VMEM_SHARED((NP,), jnp.float32),
    ],
)


def _prep_body(da_ref, db_ref, h0_ref, dinv_ref, ga_ref, gb_ref):
    deg = da_ref[...] + db_ref[...] + 1.0
    dinv = lax.rsqrt(deg)
    dinv_ref[...] = dinv
    g = dinv * h0_ref[...]
    ga_ref[...] = g[:, :HALF]
    gb_ref[...] = g[:, HALF:]


def _prep(dega, degb, h0p):
    return pl.pallas_call(
        _prep_body,
        grid=(TC_GRID,),
        in_specs=[
            pl.BlockSpec((BLK, 1), lambda i: (i, 0)),
            pl.BlockSpec((BLK, 1), lambda i: (i, 0)),
            pl.BlockSpec((BLK, EMB), lambda i: (i, 0)),
        ],
        out_specs=[
            pl.BlockSpec((BLK, 1), lambda i: (i, 0)),
            pl.BlockSpec((BLK, HALF), lambda i: (i, 0)),
            pl.BlockSpec((BLK, HALF), lambda i: (i, 0)),
        ],
        out_shape=[
            jax.ShapeDtypeStruct((NP, 1), jnp.float32),
            jax.ShapeDtypeStruct((NP, HALF), jnp.float32),
            jax.ShapeDtypeStruct((NP, HALF), jnp.float32),
        ],
    )(dega, degb, h0p)


def _hop_body(ga, gb, src_hbm, dst_hbm, outa, outb,
              idx_s, idx_d, rows, acc, sem):
    c = lax.axis_index("c")
    s = lax.axis_index("s")
    base_r = s * NODE_PER_TILE

    def _init_chunk(r0, nrow):
        @pl.when(c == 0)
        def _():
            pltpu.sync_copy(ga.at[pl.ds(r0, nrow)], rows.at[pl.ds(0, nrow)])

        @pl.when(c == 1)
        def _():
            pltpu.sync_copy(gb.at[pl.ds(r0, nrow)], rows.at[pl.ds(0, nrow)])

        pltpu.sync_copy(rows.at[pl.ds(0, nrow)], acc.at[pl.ds(r0, nrow)])

    def _init(i, _):
        _init_chunk(base_r + i * 1024, 1024)
        return _
    lax.fori_loop(0, 6, _init, None)
    _init_chunk(base_r + 6144, 128)
    plsc.subcore_barrier()

    erow0 = s * ROWS_PER_TILE

    def _grp(g, _):
        r0 = erow0 + g * 8
        pltpu.sync_copy(src_hbm.at[pl.ds(r0, 8)], idx_s)
        pltpu.sync_copy(dst_hbm.at[pl.ds(r0, 8)], idx_d)

        @pl.when(c == 0)
        def _():
            cps = [pltpu.async_copy(ga.at[idx_s.at[j]],
                                    rows.at[pl.ds(j * 128, 128)], sem)
                   for j in range(8)]
            for cp in cps:
                cp.wait()

        @pl.when(c == 1)
        def _():
            cps = [pltpu.async_copy(gb.at[idx_s.at[j]],
                                    rows.at[pl.ds(j * 128, 128)], sem)
                   for j in range(8)]
            for cp in cps:
                cp.wait()

        for j in range(8):
            pltpu.sync_copy(rows.at[pl.ds(j * 128, 128)],
                            acc.at[idx_d.at[j]], add=True)
        return _
    lax.fori_loop(0, GROUPS, _grp, None)
    plsc.subcore_barrier()

    def _out_chunk(r0, nrow):
        pltpu.sync_copy(acc.at[pl.ds(r0, nrow)], rows.at[pl.ds(0, nrow)])

        @pl.when(c == 0)
        def _():
            pltpu.sync_copy(rows.at[pl.ds(0, nrow)], outa.at[pl.ds(r0, nrow)])

        @pl.when(c == 1)
        def _():
            pltpu.sync_copy(rows.at[pl.ds(0, nrow)], outb.at[pl.ds(r0, nrow)])

    def _out(i, _):
        _out_chunk(base_r + i * 1024, 1024)
        return _
    lax.fori_loop(0, 6, _out, None)
    _out_chunk(base_r + 6144, 128)


_hop_call = pl.kernel(
    _hop_body,
    out_type=[jax.ShapeDtypeStruct((NP, HALF), jnp.float32),
              jax.ShapeDtypeStruct((NP, HALF), jnp.float32)],
    mesh=_mesh,
    scratch_types=[
        pltpu.VMEM((8, 128), jnp.int32),
        pltpu.VMEM((8, 128), jnp.int32),
        pltpu.VMEM((1024, HALF), jnp.float32),
        pltpu.VMEM_SHARED((NP, HALF), jnp.float32),
        pltpu.SemaphoreType.DMA,
    ],
    compiler_params=pltpu.CompilerParams(use_tc_tiling_on_sc=False),
)


def _combine_body(a_ref, b_ref, dinv_ref, h_ref, ga_ref, gb_ref):
    dinv = dinv_ref[...]
    ha = dinv * a_ref[...]
    hb = dinv * b_ref[...]
    h_ref[...] = jnp.concatenate([ha, hb], axis=1)
    ga_ref[...] = dinv * ha
    gb_ref[...] = dinv * hb


def _combine(accA, accB, dinv):
    return pl.pallas_call(
        _combine_body,
        grid=(TC_GRID,),
        in_specs=[
            pl.BlockSpec((BLK, HALF), lambda i: (i, 0)),
            pl.BlockSpec((BLK, HALF), lambda i: (i, 0)),
            pl.BlockSpec((BLK, 1), lambda i: (i, 0)),
        ],
        out_specs=[
            pl.BlockSpec((BLK, EMB), lambda i: (i, 0)),
            pl.BlockSpec((BLK, HALF), lambda i: (i, 0)),
            pl.BlockSpec((BLK, HALF), lambda i: (i, 0)),
        ],
        out_shape=[
            jax.ShapeDtypeStruct((NP, EMB), jnp.float32),
            jax.ShapeDtypeStruct((NP, HALF), jnp.float32),
            jax.ShapeDtypeStruct((NP, HALF), jnp.float32),
        ],
    )(accA, accB, dinv)


def _final_body(a_ref, b_ref, dinv_ref, h_ref):
    dinv = dinv_ref[...]
    h_ref[...] = jnp.concatenate([dinv * a_ref[...], dinv * b_ref[...]], axis=1)


def _final(accA, accB, dinv):
    return pl.pallas_call(
        _final_body,
        grid=(TC_GRID,),
        in_specs=[
            pl.BlockSpec((BLK, HALF), lambda i: (i, 0)),
            pl.BlockSpec((BLK, HALF), lambda i: (i, 0)),
            pl.BlockSpec((BLK, 1), lambda i: (i, 0)),
        ],
        out_specs=pl.BlockSpec((BLK, EMB), lambda i: (i, 0)),
        out_shape=jax.ShapeDtypeStruct((NP, EMB), jnp.float32),
    )(accA, accB, dinv)


def kernel(x, edge_index, W1, b1, ln_g, ln_b, W2, b2):
    src = edge_index[0].astype(jnp.int32)
    dst = edge_index[1].astype(jnp.int32)
    pad_idx = (N + (jnp.arange(EP - E, dtype=jnp.int32) % 256))
    srcp = jnp.concatenate([src, pad_idx]).reshape(ER, 128)
    dstp = jnp.concatenate([dst, pad_idx]).reshape(ER, 128)
    xp = jnp.pad(x, ((0, NP - N), (0, 0)))

    h0p = _encoder(xp, W1, b1, ln_g, ln_b, W2, b2)
    degf = _deg_call(dstp)
    dinv, g0a, g0b = _prep(degf[:NP].reshape(NP, 1),
                           degf[NP:].reshape(NP, 1), h0p)
    accA, accB = _hop_call(g0a, g0b, srcp, dstp)
    h1p, g1a, g1b = _combine(accA, accB, dinv)
    accA2, accB2 = _hop_call(g1a, g1b, srcp, dstp)
    h2p = _final(accA2, accB2, dinv)
    return (h0p[:N], h1p[:N], h2p[:N])


# speedup vs baseline: 31.7913x; 1.2783x over previous
import jax
import jax.numpy as jnp
from jax import lax
from jax.experimental import pallas as pl
from jax.experimental.pallas import tpu as pltpu
from jax.experimental.pallas import tpu_sc as plsc

N = 100000
E = 1600000
IN_CH = 128
HID = 32
EMB = 32
HALF = 16

NC = 2
NS = 16
NP = 100352
EP = 1605632
ER = EP // 128
ROWS_PER_TILE = ER // NS
GROUPS = ROWS_PER_TILE // 4
DEG_ROWS_PER_TILE = ER // (NC * NS)
DEG_GROUPS = DEG_ROWS_PER_TILE // 4
NODE_PER_TILE = NP // NS

BLK = 3136
TC_GRID = NP // BLK

_mesh = plsc.VectorSubcoreMesh(core_axis_name="c", subcore_axis_name="s")


def _encoder_body(x_ref, w1_ref, b1_ref, lg_ref, lb_ref, w2_ref, b2_ref, o_ref):
    x = x_ref[...]
    h = lax.dot_general(x, w1_ref[...], (((1,), (1,)), ((), ())),
                        preferred_element_type=jnp.float32,
                        precision=lax.Precision.HIGHEST)
    h = h + b1_ref[...]
    mu = jnp.mean(h, axis=-1, keepdims=True)
    var = jnp.mean((h - mu) ** 2, axis=-1, keepdims=True)
    h = (h - mu) * lax.rsqrt(var + 1e-06) * lg_ref[...] + lb_ref[...]
    h = jnp.maximum(h, 0.0)
    h = lax.dot_general(h, w2_ref[...], (((1,), (1,)), ((), ())),
                        preferred_element_type=jnp.float32,
                        precision=lax.Precision.HIGHEST)
    h = h + b2_ref[...]
    nrm = jnp.sqrt(jnp.sum(h * h, axis=-1, keepdims=True))
    o_ref[...] = h / jnp.maximum(nrm, 1e-12)


def _encoder(xp, W1, b1, ln_g, ln_b, W2, b2):
    full = lambda i: (0, 0)
    return pl.pallas_call(
        _encoder_body,
        grid=(TC_GRID,),
        in_specs=[
            pl.BlockSpec((BLK, IN_CH), lambda i: (i, 0)),
            pl.BlockSpec((HID, IN_CH), full),
            pl.BlockSpec((1, HID), full),
            pl.BlockSpec((1, HID), full),
            pl.BlockSpec((1, HID), full),
            pl.BlockSpec((EMB, HID), full),
            pl.BlockSpec((1, EMB), full),
        ],
        out_specs=pl.BlockSpec((BLK, EMB), lambda i: (i, 0)),
        out_shape=jax.ShapeDtypeStruct((NP, EMB), jnp.float32),
    )(xp, W1, b1.reshape(1, -1), ln_g.reshape(1, -1), ln_b.reshape(1, -1),
      W2, b2.reshape(1, -1))


def _deg_body(comb, out_hbm, idx_b, ones_v, stage, deg_acc):
    c = lax.axis_index("c")
    s = lax.axis_index("s")
    one16 = jnp.ones((16,), jnp.float32)
    zero16 = jnp.zeros((16,), jnp.float32)
    for i in range(8):
        ones_v[pl.ds(i * 16, 16)] = one16

    def _z(i, _):
        stage[pl.ds(i * 16, 16)] = zero16
        return _
    lax.fori_loop(0, NODE_PER_TILE // 16, _z, None)
    pltpu.sync_copy(stage, deg_acc.at[pl.ds(s * NODE_PER_TILE, NODE_PER_TILE)])
    plsc.subcore_barrier()

    row0 = c * (ER // 2) + s * DEG_ROWS_PER_TILE

    def _grp(g, _):
        pltpu.sync_copy(comb.at[pl.ds(row0 + g * 4, 4)], idx_b)
        for j in range(4):
            pltpu.sync_copy(ones_v.at[pl.ds(0, 128)],
                            deg_acc.at[idx_b.at[j, 1]], add=True)
        return _
    lax.fori_loop(0, DEG_GROUPS, _grp, None)
    plsc.subcore_barrier()

    off = s * NODE_PER_TILE
    pltpu.sync_copy(deg_acc.at[pl.ds(off, NODE_PER_TILE)], stage)
    pltpu.sync_copy(stage, out_hbm.at[pl.ds(c * NP + off, NODE_PER_TILE)])


_deg_call = pl.kernel(
    _deg_body,
    out_type=jax.ShapeDtypeStruct((NC * NP,), jnp.float32),
    mesh=_mesh,
    scratch_types=[
        pltpu.VMEM((4, 2, 128), jnp.int32),
        pltpu.VMEM((128,), jnp.float32),
        pltpu.VMEM((NODE_PER_TILE,), jnp.float32),
        pltpu.VMEM_SHARED((NP,), jnp.float32),
    ],
    compiler_params=pltpu.CompilerParams(use_tc_tiling_on_sc=False),
)


def _prep_body(da_ref, db_ref, h0_ref, dinv_ref, gs_ref):
    deg = da_ref[...] + db_ref[...] + 1.0
    dinv = lax.rsqrt(deg)
    dinv_ref[...] = dinv
    g = dinv * h0_ref[...]
    gs_ref[0] = g[:, :HALF]
    gs_ref[1] = g[:, HALF:]


def _prep(dega, degb, h0p):
    return pl.pallas_call(
        _prep_body,
        grid=(TC_GRID,),
        in_specs=[
            pl.BlockSpec((BLK, 1), lambda i: (i, 0)),
            pl.BlockSpec((BLK, 1), lambda i: (i, 0)),
            pl.BlockSpec((BLK, EMB), lambda i: (i, 0)),
        ],
        out_specs=[
            pl.BlockSpec((BLK, 1), lambda i: (i, 0)),
            pl.BlockSpec((NC, BLK, HALF), lambda i: (0, i, 0)),
        ],
        out_shape=[
            jax.ShapeDtypeStruct((NP, 1), jnp.float32),
            jax.ShapeDtypeStruct((NC, NP, HALF), jnp.float32),
        ],
    )(dega, degb, h0p)


def _gfire(tab, X, rows, sem):
    for j in range(4):
        pltpu.async_copy(tab.at[X.at[j, 0]], rows.at[pl.ds(j * 128, 128)], sem)


def _gwait(tab, X, rows, sem):
    for j in range(4):
        pltpu.make_async_copy(tab.at[X.at[j, 0]],
                              rows.at[pl.ds(j * 128, 128)], sem).wait()


def _sfire(acc, X, rows, sem):
    for j in range(4):
        pltpu.async_copy(rows.at[pl.ds(j * 128, 128)], acc.at[X.at[j, 1]],
                         sem, add=True)


def _swait(acc, X, rows, sem):
    for j in range(4):
        pltpu.make_async_copy(rows.at[pl.ds(j * 128, 128)],
                              acc.at[X.at[j, 1]], sem).wait()


def _hop_body(tabs, comb, out,
              X0, X1, X2, rows0, rows1, rows2, acc,
              sg0, sg1, sg2, ss0, ss1, ss2):
    c = lax.axis_index("c")
    s = lax.axis_index("s")
    tab = tabs.at[c]
    base_r = s * NODE_PER_TILE
    X = (X0, X1, X2)
    R = (rows0, rows1, rows2)
    SG = (sg0, sg1, sg2)
    SS = (ss0, ss1, ss2)

    def _icp(r0, nrow):
        pltpu.sync_copy(tab.at[pl.ds(r0, nrow)], rows0.at[pl.ds(0, nrow)])
        pltpu.sync_copy(rows0.at[pl.ds(0, nrow)], acc.at[pl.ds(r0, nrow)])

    def _init(i, _):
        _icp(base_r + i * 512, 512)
        return _
    lax.fori_loop(0, 12, _init, None)
    _icp(base_r + 12 * 512, 128)
    plsc.subcore_barrier()

    erow0 = s * ROWS_PER_TILE

    def _load(b, k):
        pltpu.sync_copy(comb.at[pl.ds(erow0 + 4 * k, 4)], X[b])

    _load(0, 0)
    _gfire(tab, X[0], R[0], SG[0])

    def _half(k, b, guard):
        bp = (b + 2) % 3
        if guard is None:
            _swait(acc, X[b], R[b], SS[b])
        else:
            @pl.when(guard)
            def _():
                _swait(acc, X[b], R[b], SS[b])
        _load(b, k)
        _gfire(tab, X[b], R[b], SG[b])
        _gwait(tab, X[bp], R[bp], SG[bp])
        _sfire(acc, X[bp], R[bp], SS[bp])

    def _step(i, _):
        _half(3 * i - 2, 1, i >= 2)
        _half(3 * i - 1, 2, i >= 2)
        _half(3 * i, 0, None)
        return _
    lax.fori_loop(1, 66, _step, None)

    _gwait(tab, X[0], R[0], SG[0])
    _sfire(acc, X[0], R[0], SS[0])
    _swait(acc, X[1], R[1], SS[1])
    _swait(acc, X[2], R[2], SS[2])
    _swait(acc, X[0], R[0], SS[0])
    plsc.subcore_barrier()

    def _ocp(r0, nrow):
        pltpu.sync_copy(acc.at[pl.ds(r0, nrow)], rows0.at[pl.ds(0, nrow)])
        pltpu.sync_copy(rows0.at[pl.ds(0, nrow)], out.at[c, pl.ds(r0, nrow)])

    def _out(i, _):
        _ocp(base_r + i * 512, 512)
        return _
    lax.fori_loop(0, 12, _out, None)
    _ocp(base_r + 12 * 512, 128)


_hop_call = pl.kernel(
    _hop_body,
    out_type=jax.ShapeDtypeStruct((NC, NP, HALF), jnp.float32),
    mesh=_mesh,
    scratch_types=[
        pltpu.VMEM((4, 2, 128), jnp.int32),
        pltpu.VMEM((4, 2, 128), jnp.int32),
        pltpu.VMEM((4, 2, 128), jnp.int32),
        pltpu.VMEM((512, HALF), jnp.float32),
        pltpu.VMEM((512, HALF), jnp.float32),
        pltpu.VMEM((512, HALF), jnp.float32),
        pltpu.VMEM_SHARED((NP, HALF), jnp.float32),
        pltpu.SemaphoreType.DMA,
        pltpu.SemaphoreType.DMA,
        pltpu.SemaphoreType.DMA,
        pltpu.SemaphoreType.DMA,
        pltpu.SemaphoreType.DMA,
        pltpu.SemaphoreType.DMA,
    ],
    compiler_params=pltpu.CompilerParams(use_tc_tiling_on_sc=False),
)


def _combine_body(acc_ref, dinv_ref, h_ref, gs_ref):
    dinv = dinv_ref[...]
    ha = dinv * acc_ref[0]
    hb = dinv * acc_ref[1]
    h_ref[...] = jnp.concatenate([ha, hb], axis=1)
    gs_ref[0] = dinv * ha
    gs_ref[1] = dinv * hb


def _combine(accs, dinv):
    return pl.pallas_call(
        _combine_body,
        grid=(TC_GRID,),
        in_specs=[
            pl.BlockSpec((NC, BLK, HALF), lambda i: (0, i, 0)),
            pl.BlockSpec((BLK, 1), lambda i: (i, 0)),
        ],
        out_specs=[
            pl.BlockSpec((BLK, EMB), lambda i: (i, 0)),
            pl.BlockSpec((NC, BLK, HALF), lambda i: (0, i, 0)),
        ],
        out_shape=[
            jax.ShapeDtypeStruct((NP, EMB), jnp.float32),
            jax.ShapeDtypeStruct((NC, NP, HALF), jnp.float32),
        ],
    )(accs, dinv)


def _final_body(acc_ref, dinv_ref, h_ref):
    dinv = dinv_ref[...]
    h_ref[...] = jnp.concatenate([dinv * acc_ref[0], dinv * acc_ref[1]],
                                 axis=1)


def _final(accs, dinv):
    return pl.pallas_call(
        _final_body,
        grid=(TC_GRID,),
        in_specs=[
            pl.BlockSpec((NC, BLK, HALF), lambda i: (0, i, 0)),
            pl.BlockSpec((BLK, 1), lambda i: (i, 0)),
        ],
        out_specs=pl.BlockSpec((BLK, EMB), lambda i: (i, 0)),
        out_shape=jax.ShapeDtypeStruct((NP, EMB), jnp.float32),
    )(accs, dinv)


def kernel(x, edge_index, W1, b1, ln_g, ln_b, W2, b2):
    src = edge_index[0].astype(jnp.int32)
    dst = edge_index[1].astype(jnp.int32)
    pad_idx = (N + (jnp.arange(EP - E, dtype=jnp.int32) % 256))
    srcp = jnp.concatenate([src, pad_idx]).reshape(ER, 128)
    dstp = jnp.concatenate([dst, pad_idx]).reshape(ER, 128)
    comb = jnp.stack([srcp, dstp], axis=1)
    xp = jnp.pad(x, ((0, NP - N), (0, 0)))

    h0p = _encoder(xp, W1, b1, ln_g, ln_b, W2, b2)
    degf = _deg_call(comb)
    dinv, gs0 = _prep(degf[:NP].reshape(NP, 1), degf[NP:].reshape(NP, 1), h0p)
    accs = _hop_call(gs0, comb)
    h1p, gs1 = _combine(accs, dinv)
    accs2 = _hop_call(gs1, comb)
    h2p = _final(accs2, dinv)
    return (h0p[:N], h1p[:N], h2p[:N])
